# Initial kernel scaffold; baseline (speedup 1.0000x reference)
import jax
import jax.numpy as jnp
from jax.experimental import pallas as pl


def kernel(query_embeddings, similarity_weights, W_struct, b_struct, W_sem, b_sem, W_stat, b_stat, W_cont, b_cont):
    raise NotImplementedError("write your pallas kernel here")


# speedup vs baseline: 13.0341x; 13.0341x over previous
import jax
import jax.numpy as jnp
from jax import lax
from jax.experimental import pallas as pl

N = 4096
D = 128
K = 50
RB = 256
NB = N // RB
BS_ITERS = 48


def _dot_t(a, b):
    return lax.dot_general(a, b, (((1,), (1,)), ((), ())),
                           preferred_element_type=jnp.float32)


def _emb_kernel(w_ref, q_ref, ws_ref, bs_ref, wm_ref, bm_ref,
                wt_ref, bt_ref, wc_ref, bc_ref, out_ref):
    q = q_ref[...]
    structural = _dot_t(q, ws_ref[...]) + bs_ref[...]
    semantic = _dot_t(q, wm_ref[...]) + bm_ref[...]
    statistical = _dot_t(q, wt_ref[...]) + bt_ref[...]
    content = _dot_t(q, wc_ref[...]) + bc_ref[...]
    w = w_ref[...]
    weighted = (w[0:1, 0:1] * structural + w[0:1, 1:2] * semantic
                + w[0:1, 2:3] * statistical + w[0:1, 3:4] * content)
    nrm = jnp.sqrt(jnp.sum(weighted * weighted, axis=1, keepdims=True))
    out_ref[...] = weighted / jnp.maximum(nrm, 1e-12)


def _thresh_kernel(embb_ref, emb_ref, tsub_ref, tlane_ref):
    s = _dot_t(embb_ref[...], emb_ref[...])

    def body(_, carry):
        lo, hi = carry
        mid = 0.5 * (lo + hi)
        cnt = jnp.sum((s >= mid).astype(jnp.float32), axis=1, keepdims=True)
        pred = cnt >= K
        return jnp.where(pred, mid, lo), jnp.where(pred, hi, mid)

    lo0 = jnp.full((RB, 1), -1.02, dtype=jnp.float32)
    hi0 = jnp.full((RB, 1), 1.02, dtype=jnp.float32)
    lo, _ = lax.fori_loop(0, BS_ITERS, body, (lo0, hi0))
    tsub_ref[...] = lo
    tlane_ref[...] = lo.T


def _fill_kernel(embb_ref, emb_ref, tsub_ref, tlane_ref, out_ref):
    i = pl.program_id(0)
    s = _dot_t(embb_ref[...], emb_ref[...])
    keep = (s >= tsub_ref[...]) | (s >= tlane_ref[...])
    sm = jnp.where(keep, s, 0.0)
    rows = i * RB + lax.broadcasted_iota(jnp.int32, (RB, N), 0)
    cols = lax.broadcasted_iota(jnp.int32, (RB, N), 1)
    out_ref[...] = jnp.where(rows == cols, 1.0, sm)


@jax.jit
def kernel(query_embeddings, similarity_weights, W_struct, b_struct,
           W_sem, b_sem, W_stat, b_stat, W_cont, b_cont):
    w = jax.nn.softmax(similarity_weights, axis=0).reshape(1, 4)
    biases = [b.reshape(1, D) for b in (b_struct, b_sem, b_stat, b_cont)]

    emb = pl.pallas_call(
        _emb_kernel,
        out_shape=jax.ShapeDtypeStruct((N, D), jnp.float32),
    )(w, query_embeddings, W_struct, biases[0], W_sem, biases[1],
      W_stat, biases[2], W_cont, biases[3])

    blk_spec = pl.BlockSpec((RB, D), lambda i: (i, 0))
    full_spec = pl.BlockSpec((N, D), lambda i: (0, 0))
    t_sub, t_lane = pl.pallas_call(
        _thresh_kernel,
        grid=(NB,),
        in_specs=[blk_spec, full_spec],
        out_specs=[pl.BlockSpec((RB, 1), lambda i: (i, 0)),
                   pl.BlockSpec((1, RB), lambda i: (0, i))],
        out_shape=[jax.ShapeDtypeStruct((N, 1), jnp.float32),
                   jax.ShapeDtypeStruct((1, N), jnp.float32)],
    )(emb, emb)

    M = pl.pallas_call(
        _fill_kernel,
        grid=(NB,),
        in_specs=[blk_spec, full_spec,
                  pl.BlockSpec((RB, 1), lambda i: (i, 0)),
                  pl.BlockSpec((1, N), lambda i: (0, 0))],
        out_specs=pl.BlockSpec((RB, N), lambda i: (i, 0)),
        out_shape=jax.ShapeDtypeStruct((N, N), jnp.float32),
    )(emb, emb, t_sub, t_lane)
    return M


# speedup vs baseline: 17.6005x; 1.3503x over previous
import jax
import jax.numpy as jnp
from jax import lax
from jax.experimental import pallas as pl

N = 4096
D = 128
K = 50
RB = 256
NB = N // RB
KEY_HI = 0x3F8147AE
BS_ITERS = 31


def _keys(s):
    k = lax.bitcast_convert_type(s, jnp.int32)
    return jnp.where(k >= 0, k, k ^ jnp.int32(0x7FFFFFFF))


def _dot_t(a, b):
    return lax.dot_general(a, b, (((1,), (1,)), ((), ())),
                           preferred_element_type=jnp.float32)


def _emb_kernel(w_ref, q_ref, ws_ref, bs_ref, wm_ref, bm_ref,
                wt_ref, bt_ref, wc_ref, bc_ref, out_ref):
    q = q_ref[...]
    structural = _dot_t(q, ws_ref[...]) + bs_ref[...]
    semantic = _dot_t(q, wm_ref[...]) + bm_ref[...]
    statistical = _dot_t(q, wt_ref[...]) + bt_ref[...]
    content = _dot_t(q, wc_ref[...]) + bc_ref[...]
    w = w_ref[...]
    weighted = (w[0:1, 0:1] * structural + w[0:1, 1:2] * semantic
                + w[0:1, 2:3] * statistical + w[0:1, 3:4] * content)
    nrm = jnp.sqrt(jnp.sum(weighted * weighted, axis=1, keepdims=True))
    out_ref[...] = weighted / jnp.maximum(nrm, 1e-12)


def _thresh_kernel(embb_ref, emb_ref, tsub_ref, tlane_ref):
    k = _keys(_dot_t(embb_ref[...], emb_ref[...]))

    def body(_, carry):
        lo, hi = carry
        mid = lax.shift_right_arithmetic(lo + hi, 1)
        cnt = jnp.sum((k >= mid).astype(jnp.int32), axis=1, keepdims=True)
        pred = cnt >= K
        return jnp.where(pred, mid, lo), jnp.where(pred, hi, mid)

    lo0 = jnp.full((RB, 1), -(KEY_HI + 1), dtype=jnp.int32)
    hi0 = jnp.full((RB, 1), KEY_HI, dtype=jnp.int32)
    lo, _ = lax.fori_loop(0, BS_ITERS, body, (lo0, hi0))
    tsub_ref[...] = lo
    tlane_ref[...] = lo.T


def _fill_kernel(embb_ref, emb_ref, tsub_ref, tlane_ref, out_ref):
    i = pl.program_id(0)
    s = _dot_t(embb_ref[...], emb_ref[...])
    k = _keys(s)
    keep = (k >= tsub_ref[...]) | (k >= tlane_ref[...])
    sm = jnp.where(keep, s, 0.0)
    rows = i * RB + lax.broadcasted_iota(jnp.int32, (RB, N), 0)
    cols = lax.broadcasted_iota(jnp.int32, (RB, N), 1)
    out_ref[...] = jnp.where(rows == cols, 1.0, sm)


@jax.jit
def kernel(query_embeddings, similarity_weights, W_struct, b_struct,
           W_sem, b_sem, W_stat, b_stat, W_cont, b_cont):
    w = jax.nn.softmax(similarity_weights, axis=0).reshape(1, 4)
    biases = [b.reshape(1, D) for b in (b_struct, b_sem, b_stat, b_cont)]

    emb = pl.pallas_call(
        _emb_kernel,
        out_shape=jax.ShapeDtypeStruct((N, D), jnp.float32),
    )(w, query_embeddings, W_struct, biases[0], W_sem, biases[1],
      W_stat, biases[2], W_cont, biases[3])

    blk_spec = pl.BlockSpec((RB, D), lambda i: (i, 0))
    full_spec = pl.BlockSpec((N, D), lambda i: (0, 0))
    t_sub, t_lane = pl.pallas_call(
        _thresh_kernel,
        grid=(NB,),
        in_specs=[blk_spec, full_spec],
        out_specs=[pl.BlockSpec((RB, 1), lambda i: (i, 0)),
                   pl.BlockSpec((1, RB), lambda i: (0, i))],
        out_shape=[jax.ShapeDtypeStruct((N, 1), jnp.int32),
                   jax.ShapeDtypeStruct((1, N), jnp.int32)],
    )(emb, emb)

    M = pl.pallas_call(
        _fill_kernel,
        grid=(NB,),
        in_specs=[blk_spec, full_spec,
                  pl.BlockSpec((RB, 1), lambda i: (i, 0)),
                  pl.BlockSpec((1, N), lambda i: (0, 0))],
        out_specs=pl.BlockSpec((RB, N), lambda i: (i, 0)),
        out_shape=jax.ShapeDtypeStruct((N, N), jnp.float32),
    )(emb, emb, t_sub, t_lane)
    return M
